# baseline (speedup 1.0000x reference)
import functools

import jax
import jax.numpy as jnp
from jax import lax
from jax.experimental import pallas as pl
from jax.experimental.pallas import tpu as pltpu
from jax.experimental.pallas import tpu_sc as plsc

_B = 8
_S = 512
_D = 512
_MAX_REL = 32
_VOCAB = 2 * _MAX_REL + 1
_NC = 2
_NS = 16
_NW = _NC * _NS
_ROWS = _S // _NW
_L = 16
_NCH = _D // _L


def _sc_body(x_hbm, pe_hbm, out_hbm, pe_v, pe64_v, rowpe_v, x_v):
    wid = lax.axis_index("s") * _NC + lax.axis_index("c")
    base = wid * _ROWS

    pltpu.sync_copy(pe_hbm, pe_v)

    def _save64(c, carry):
        sl = pl.ds(c * _L, _L)
        pe64_v[0, sl] = pe_v[_VOCAB - 1, sl]
        return carry

    lax.fori_loop(0, _NCH, _save64, 0)

    def _cum_m(m, carry):
        def _cum_c(c, carry2):
            sl = pl.ds(c * _L, _L)
            pe_v[m, sl] = pe_v[m, sl] + pe_v[m - 1, sl]
            return carry2

        return lax.fori_loop(0, _NCH, _cum_c, carry)

    lax.fori_loop(1, _VOCAB, _cum_m, 0)

    inv = jnp.float32(1.0 / _S)

    def _row_t(t, carry):
        i = base + t
        a = jnp.maximum(_S - _MAX_REL - i, 0).astype(jnp.float32)
        b = jnp.maximum(i - (_MAX_REL - 1), 0).astype(jnp.float32)
        hi = jnp.minimum(i + _MAX_REL, _VOCAB - 2)
        lo = jnp.maximum(i - (_S - _MAX_REL), 0)

        def _row_c(c, carry2):
            sl = pl.ds(c * _L, _L)
            val = (
                a * pe_v[0, sl]
                + b * pe64_v[0, sl]
                + pe_v[hi, sl]
                - pe_v[lo, sl]
            ) * inv
            rowpe_v[t, sl] = val
            return carry2

        return lax.fori_loop(0, _NCH, _row_c, carry)

    lax.fori_loop(0, _ROWS, _row_t, 0)

    def _batch(b_i, carry):
        pltpu.sync_copy(x_hbm.at[b_i, pl.ds(base, _ROWS)], x_v)

        def _add_t(t, carry2):
            def _add_c(c, carry3):
                sl = pl.ds(c * _L, _L)
                x_v[t, sl] = x_v[t, sl] + rowpe_v[t, sl]
                return carry3

            return lax.fori_loop(0, _NCH, _add_c, carry2)

        lax.fori_loop(0, _ROWS, _add_t, 0)
        pltpu.sync_copy(x_v, out_hbm.at[b_i, pl.ds(base, _ROWS)])
        return carry

    lax.fori_loop(0, _B, _batch, 0)


@jax.jit
def kernel(x, pe):
    b, s, d = x.shape
    mesh = plsc.VectorSubcoreMesh(
        core_axis_name="c", subcore_axis_name="s",
        num_cores=_NC, num_subcores=_NS,
    )
    sc_call = pl.kernel(
        _sc_body,
        out_type=jax.ShapeDtypeStruct((b, s, d), jnp.float32),
        mesh=mesh,
        scratch_types=[
            pltpu.VMEM((_VOCAB, d), jnp.float32),
            pltpu.VMEM((1, d), jnp.float32),
            pltpu.VMEM((_ROWS, d), jnp.float32),
            pltpu.VMEM((_ROWS, d), jnp.float32),
        ],
    )
    return sc_call(x, pe)
